# Initial kernel scaffold; baseline (speedup 1.0000x reference)
import jax
import jax.numpy as jnp
from jax.experimental import pallas as pl


def kernel(indices, tables):
    raise NotImplementedError("write your pallas kernel here")


# speedup vs baseline: 3.0968x; 3.0968x over previous
import functools

import jax
import jax.numpy as jnp
from jax import lax
from jax.experimental import pallas as pl
from jax.experimental.pallas import tpu as pltpu
from jax.experimental.pallas import tpu_sc as plsc

B, F, L, V, D = 4096, 26, 20, 100000, 32
NC, NS = 2, 16
NW = NC * NS
BAGS = B * F
BAGS_W = BAGS // NW
CB = 32
ROWS_CB = CB * L
NCHUNK = BAGS_W // CB
GW = 128
NG = ROWS_CB // GW


def _sc_pooled_lookup(tbl_hbm, idx_hbm, out_hbm,
                      idx_v, rows0, rows1, out_v,
                      sg0, sg1, so0, so1):
    wid = lax.axis_index("s") * NC + lax.axis_index("c")

    pltpu.sync_copy(idx_hbm.at[wid], idx_v)

    rows = (rows0, rows1)
    sg = (sg0, sg1)
    so = (so0, so1)

    def gather_start(cc, b):
        for g in range(NG):
            pltpu.make_async_copy(tbl_hbm.at[idx_v.at[cc * NG + g]],
                                  rows[b].at[pl.ds(g * GW, GW)],
                                  sg[b]).start()

    def gather_wait(cc, b):
        for g in range(NG):
            pltpu.make_async_copy(tbl_hbm.at[idx_v.at[cc * NG + g]],
                                  rows[b].at[pl.ds(g * GW, GW)],
                                  sg[b]).wait()

    def reduce_chunk(b):
        rv = rows[b]

        def bag_body(bag, _):
            r = bag * L
            a0 = rv[r, pl.ds(0, 16)]
            a1 = rv[r, pl.ds(16, 16)]
            for l in range(1, L):
                a0 = a0 + rv[r + l, pl.ds(0, 16)]
                a1 = a1 + rv[r + l, pl.ds(16, 16)]
            out_v[b, bag, pl.ds(0, 16)] = a0
            out_v[b, bag, pl.ds(16, 16)] = a1
            return _

        lax.fori_loop(0, CB, bag_body, 0, unroll=False)

    def store_start(cc, b):
        base = wid * BAGS_W + cc * CB
        pltpu.make_async_copy(out_v.at[b], out_hbm.at[pl.ds(base, CB)],
                              so[b]).start()

    def store_wait(cc, b):
        base = wid * BAGS_W + cc * CB
        pltpu.make_async_copy(out_v.at[b], out_hbm.at[pl.ds(base, CB)],
                              so[b]).wait()

    gather_start(0, 0)
    gather_start(1, 1)
    for b in range(2):
        gather_wait(b, b)
        reduce_chunk(b)
        store_start(b, b)
        gather_start(b + 2, b)

    def step(i, _):
        j = 2 + 2 * i
        for b in range(2):
            cc = j + b
            gather_wait(cc, b)
            store_wait(cc - 2, b)
            reduce_chunk(b)
            store_start(cc, b)
            gather_start(cc + 2, b)
        return _

    lax.fori_loop(0, (NCHUNK - 4) // 2, step, 0, unroll=False)

    for b in range(2):
        cc = NCHUNK - 2 + b
        gather_wait(cc, b)
        store_wait(cc - 2, b)
        reduce_chunk(b)
        store_start(cc, b)
    for b in range(2):
        store_wait(NCHUNK - 2 + b, b)


@jax.jit
def kernel(indices, tables):
    idx_g = (indices.astype(jnp.int32)
             + (jnp.arange(F, dtype=jnp.int32) * V)[None, :, None])
    idx_g = idx_g.reshape(NW, NCHUNK * NG, GW)
    tbl = tables.reshape(F * V, D)

    mesh = plsc.VectorSubcoreMesh(core_axis_name="c", subcore_axis_name="s")
    run = functools.partial(
        pl.kernel,
        out_type=jax.ShapeDtypeStruct((BAGS, D), jnp.float32),
        mesh=mesh,
        compiler_params=pltpu.CompilerParams(use_tc_tiling_on_sc=False),
        scratch_types=[
            pltpu.VMEM((NCHUNK * NG, GW), jnp.int32),
            pltpu.VMEM((ROWS_CB, D), jnp.float32),
            pltpu.VMEM((ROWS_CB, D), jnp.float32),
            pltpu.VMEM((2, CB, D), jnp.float32),
            pltpu.SemaphoreType.DMA,
            pltpu.SemaphoreType.DMA,
            pltpu.SemaphoreType.DMA,
            pltpu.SemaphoreType.DMA,
        ],
    )(_sc_pooled_lookup)
    out = run(tbl, idx_g)
    return out.reshape(B, F, D)


# speedup vs baseline: 4.7757x; 1.5421x over previous
import functools

import jax
import jax.numpy as jnp
from jax import lax
from jax.experimental import pallas as pl
from jax.experimental.pallas import tpu as pltpu
from jax.experimental.pallas import tpu_sc as plsc

B, F, L, V, D = 4096, 26, 20, 100000, 32
NC, NS = 2, 16
NW = NC * NS
UNITS = F * D
UNITS_W = UNITS // NW
BB = 512
NBLK = B // BB
NGRP = BB // 16


def _sc_pooled_lookup_t(tbl_hbm, idx_hbm, out_hbm,
                        slice_v, idxv0, idxv1, outv0, outv1,
                        sem_s, sem_i0, sem_i1, sem_o0, sem_o1):
    wid = lax.axis_index("s") * NC + lax.axis_index("c")
    idxv = (idxv0, idxv1)
    sem_i = (sem_i0, sem_i1)
    outv = (outv0, outv1)
    sem_o = (sem_o0, sem_o1)

    def idx_copy(f, blk, ib):
        return pltpu.make_async_copy(
            idx_hbm.at[pl.ds(f * L, L), pl.ds(blk * BB, BB)], idxv[ib],
            sem_i[ib])

    def out_copy(u, ob):
        return pltpu.make_async_copy(outv[ob], out_hbm.at[u], sem_o[ob])

    def process_blocks(f, ob):
        idx_copy(f, 0, 0).start()

        def blk_pair(bb_i, carry):
            for ib in range(2):
                blk = 2 * bb_i + ib
                if ib == 0:
                    idx_copy(f, blk + 1, 1).start()
                else:
                    @pl.when(bb_i < NBLK // 2 - 1)
                    def _():
                        idx_copy(f, blk + 1, 0).start()
                idx_copy(f, blk, ib).wait()
                ivb = idxv[ib]
                ovb = outv[ob]

                def grp_body(grp, _g):
                    base = grp * 16
                    acc = plsc.load_gather(slice_v, [ivb[0, pl.ds(base, 16)]])
                    for l in range(1, L):
                        acc = acc + plsc.load_gather(
                            slice_v, [ivb[l, pl.ds(base, 16)]])
                    ovb[pl.ds(blk * BB + base, 16)] = acc
                    return _g

                lax.fori_loop(0, NGRP, grp_body, 0, unroll=False)
            return carry

        lax.fori_loop(0, NBLK // 2, blk_pair, 0, unroll=False)

    def unit_pair(kk, carry):
        for ob in range(2):
            k = 2 * kk + ob
            u = wid * UNITS_W + k
            f = u // D

            pltpu.make_async_copy(tbl_hbm.at[u], slice_v, sem_s).start()
            pltpu.make_async_copy(tbl_hbm.at[u], slice_v, sem_s).wait()

            @pl.when(kk >= 1)
            def _():
                out_copy(u - 2, ob).wait()

            process_blocks(f, ob)
            out_copy(u, ob).start()
        return carry

    lax.fori_loop(0, UNITS_W // 2, unit_pair, 0, unroll=False)
    last = wid * UNITS_W + UNITS_W
    out_copy(last - 2, 0).wait()
    out_copy(last - 1, 1).wait()


@jax.jit
def kernel(indices, tables):
    tbl_t = tables.transpose(0, 2, 1).reshape(UNITS, V)
    idx_t = indices.astype(jnp.int32).transpose(1, 2, 0).reshape(F * L, B)

    mesh = plsc.VectorSubcoreMesh(core_axis_name="c", subcore_axis_name="s")
    run = functools.partial(
        pl.kernel,
        out_type=jax.ShapeDtypeStruct((UNITS, B), jnp.float32),
        mesh=mesh,
        compiler_params=pltpu.CompilerParams(use_tc_tiling_on_sc=False,
                                             needs_layout_passes=False),
        scratch_types=[
            pltpu.VMEM((V,), jnp.float32),
            pltpu.VMEM((L, BB), jnp.int32),
            pltpu.VMEM((L, BB), jnp.int32),
            pltpu.VMEM((B,), jnp.float32),
            pltpu.VMEM((B,), jnp.float32),
            pltpu.SemaphoreType.DMA,
            pltpu.SemaphoreType.DMA,
            pltpu.SemaphoreType.DMA,
            pltpu.SemaphoreType.DMA,
            pltpu.SemaphoreType.DMA,
        ],
    )(_sc_pooled_lookup_t)
    out_t = run(tbl_t, idx_t)
    return out_t.reshape(F, D, B).transpose(2, 0, 1)


# speedup vs baseline: 10.0665x; 2.1079x over previous
import functools

import jax
import jax.numpy as jnp
from jax import lax
from jax.experimental import pallas as pl
from jax.experimental.pallas import tpu as pltpu
from jax.experimental.pallas import tpu_sc as plsc

B, F, L, V, D = 4096, 26, 20, 100000, 32
NC, NS = 2, 16
NW = NC * NS
UNITS = F * D
UNITS_W = UNITS // NW
BB = 512
NBLK = B // BB
NGRP = BB // 16
VMAIN = (V // 128) * 128
VTAIL = V - VMAIN


def _sc_pooled_lookup_t(tbl_hbm, tail_hbm, idx_hbm, out_hbm,
                        slice_v, idxv0, idxv1, outv,
                        sem_s, sem_i0, sem_i1, sem_o):
    wid = lax.axis_index("s") * NC + lax.axis_index("c")
    idxv = (idxv0, idxv1)
    sem_i = (sem_i0, sem_i1)

    def idx_copy(f, blk, ib):
        return pltpu.make_async_copy(
            idx_hbm.at[f, :, pl.ds(blk * BB, BB)], idxv[ib], sem_i[ib])

    def out_copy(u):
        return pltpu.make_async_copy(outv, out_hbm.at[u], sem_o)

    def slice_copies(f, d, u):
        return (
            pltpu.make_async_copy(tbl_hbm.at[f, d, pl.ds(0, VMAIN)],
                                  slice_v.at[pl.ds(0, VMAIN)], sem_s),
            pltpu.make_async_copy(tail_hbm.at[u],
                                  slice_v.at[pl.ds(VMAIN, 128)], sem_s),
        )

    def process_blocks(f):
        idx_copy(f, 0, 0).start()

        def blk_pair(bb_i, carry):
            for ib in range(2):
                blk = 2 * bb_i + ib
                if ib == 0:
                    idx_copy(f, blk + 1, 1).start()
                else:
                    @pl.when(bb_i < NBLK // 2 - 1)
                    def _start_next():
                        idx_copy(f, blk + 1, 0).start()
                idx_copy(f, blk, ib).wait()
                ivb = idxv[ib]
                ovb = outv

                def grp_body(grp, _g):
                    base = grp * 16
                    acc = plsc.load_gather(slice_v, [ivb[0, pl.ds(base, 16)]])
                    for l in range(1, L):
                        acc = acc + plsc.load_gather(
                            slice_v, [ivb[l, pl.ds(base, 16)]])
                    ovb[pl.ds(blk * BB + base, 16)] = acc
                    return _g

                lax.fori_loop(0, NGRP, grp_body, 0, unroll=2)
            return carry

        lax.fori_loop(0, NBLK // 2, blk_pair, 0, unroll=False)

    def unit_body(k, carry):
        u = wid * UNITS_W + k
        f = u // D
        d = u - f * D

        main_cp, tail_cp = slice_copies(f, d, u)
        main_cp.start()
        tail_cp.start()
        main_cp.wait()
        tail_cp.wait()

        @pl.when(k >= 1)
        def _drain_prev():
            out_copy(u - 1).wait()

        process_blocks(f)
        out_copy(u).start()
        return carry

    lax.fori_loop(0, UNITS_W, unit_body, 0, unroll=False)
    out_copy(wid * UNITS_W + UNITS_W - 1).wait()


@jax.jit
def kernel(indices, tables):
    tbl_t = tables.transpose(0, 2, 1)
    idx_t = indices.astype(jnp.int32).transpose(1, 2, 0)
    tail = jnp.pad(tbl_t[:, :, VMAIN:], ((0, 0), (0, 0), (0, 128 - VTAIL)))
    tail = tail.reshape(UNITS, 128)

    mesh = plsc.VectorSubcoreMesh(core_axis_name="c", subcore_axis_name="s")
    run = functools.partial(
        pl.kernel,
        out_type=jax.ShapeDtypeStruct((UNITS, B), jnp.float32),
        mesh=mesh,
        compiler_params=pltpu.CompilerParams(needs_layout_passes=False),
        scratch_types=[
            pltpu.VMEM((VMAIN + 128,), jnp.float32),
            pltpu.VMEM((L, BB), jnp.int32),
            pltpu.VMEM((L, BB), jnp.int32),
            pltpu.VMEM((B,), jnp.float32),
            pltpu.SemaphoreType.DMA,
            pltpu.SemaphoreType.DMA,
            pltpu.SemaphoreType.DMA,
            pltpu.SemaphoreType.DMA,
        ],
    )(_sc_pooled_lookup_t)
    out_t = run(tbl_t, tail, idx_t)
    return out_t.reshape(F, D, B).transpose(2, 0, 1)


# speedup vs baseline: 10.7739x; 1.0703x over previous
import functools

import jax
import jax.numpy as jnp
from jax import lax
from jax.experimental import pallas as pl
from jax.experimental.pallas import tpu as pltpu
from jax.experimental.pallas import tpu_sc as plsc

B, F, L, V, D = 4096, 26, 20, 100000, 32
NC, NS = 2, 16
NW = NC * NS
UNITS = F * D
UNITS_W = UNITS // NW
BB = 512
NBLK = B // BB
NGRP = BB // 16
VMAIN = (V // 128) * 128
VTAIL = V - VMAIN


def _sc_pooled_lookup_t(tbl_hbm, tail_hbm, idx_hbm, out_hbm,
                        slice_v, idxv0, idxv1, outv,
                        sem_s, sem_i0, sem_i1, sem_o):
    wid = lax.axis_index("s") * NC + lax.axis_index("c")
    idxv = (idxv0, idxv1)
    sem_i = (sem_i0, sem_i1)

    def idx_copy(f, blk, ib):
        return pltpu.make_async_copy(
            idx_hbm.at[f, :, pl.ds(blk * BB, BB)], idxv[ib], sem_i[ib])

    def out_copy(u):
        return pltpu.make_async_copy(outv, out_hbm.at[u], sem_o)

    def slice_copies(f, d, u):
        return (
            pltpu.make_async_copy(tbl_hbm.at[f, d, pl.ds(0, VMAIN)],
                                  slice_v.at[pl.ds(0, VMAIN)], sem_s),
            pltpu.make_async_copy(tail_hbm.at[u],
                                  slice_v.at[pl.ds(VMAIN, 128)], sem_s),
        )

    def process_blocks(f):
        def blk_pair(bb_i, carry):
            for ib in range(2):
                blk = 2 * bb_i + ib
                if ib == 0:
                    idx_copy(f, blk + 1, 1).start()
                else:
                    @pl.when(bb_i < NBLK // 2 - 1)
                    def _start_next():
                        idx_copy(f, blk + 1, 0).start()
                idx_copy(f, blk, ib).wait()
                ivb = idxv[ib]
                ovb = outv

                def grp_body(grp, _g):
                    base = grp * 16
                    acc0 = plsc.load_gather(slice_v, [ivb[0, pl.ds(base, 16)]])
                    acc1 = plsc.load_gather(slice_v, [ivb[1, pl.ds(base, 16)]])
                    for l in range(2, L, 2):
                        acc0 = acc0 + plsc.load_gather(
                            slice_v, [ivb[l, pl.ds(base, 16)]])
                        acc1 = acc1 + plsc.load_gather(
                            slice_v, [ivb[l + 1, pl.ds(base, 16)]])
                    ovb[pl.ds(blk * BB + base, 16)] = acc0 + acc1
                    return _g

                lax.fori_loop(0, NGRP, grp_body, 0, unroll=4)
            return carry

        lax.fori_loop(0, NBLK // 2, blk_pair, 0, unroll=False)

    def unit_body(k, carry):
        u = wid * UNITS_W + k
        f = u // D
        d = u - f * D

        main_cp, tail_cp = slice_copies(f, d, u)
        main_cp.start()
        tail_cp.start()
        idx_copy(f, 0, 0).start()

        @pl.when(k >= 1)
        def _drain_prev():
            out_copy(u - 1).wait()

        main_cp.wait()
        tail_cp.wait()
        process_blocks(f)
        out_copy(u).start()
        return carry

    lax.fori_loop(0, UNITS_W, unit_body, 0, unroll=False)
    out_copy(wid * UNITS_W + UNITS_W - 1).wait()


@jax.jit
def kernel(indices, tables):
    tbl_t = tables.transpose(0, 2, 1)
    idx_t = indices.astype(jnp.int32).transpose(1, 2, 0)
    tail = jnp.pad(tbl_t[:, :, VMAIN:], ((0, 0), (0, 0), (0, 128 - VTAIL)))
    tail = tail.reshape(UNITS, 128)

    mesh = plsc.VectorSubcoreMesh(core_axis_name="c", subcore_axis_name="s")
    run = functools.partial(
        pl.kernel,
        out_type=jax.ShapeDtypeStruct((UNITS, B), jnp.float32),
        mesh=mesh,
        compiler_params=pltpu.CompilerParams(needs_layout_passes=False),
        scratch_types=[
            pltpu.VMEM((VMAIN + 128,), jnp.float32),
            pltpu.VMEM((L, BB), jnp.int32),
            pltpu.VMEM((L, BB), jnp.int32),
            pltpu.VMEM((B,), jnp.float32),
            pltpu.SemaphoreType.DMA,
            pltpu.SemaphoreType.DMA,
            pltpu.SemaphoreType.DMA,
            pltpu.SemaphoreType.DMA,
        ],
    )(_sc_pooled_lookup_t)
    out_t = run(tbl_t, tail, idx_t)
    return out_t.reshape(F, D, B).transpose(2, 0, 1)


# speedup vs baseline: 10.7902x; 1.0015x over previous
import functools

import jax
import jax.numpy as jnp
from jax import lax
from jax.experimental import pallas as pl
from jax.experimental.pallas import tpu as pltpu
from jax.experimental.pallas import tpu_sc as plsc

B, F, L, V, D = 4096, 26, 20, 100000, 32
NC, NS = 2, 16
NW = NC * NS
UNITS = F * D
UNITS_W = UNITS // NW
BB = 512
NBLK = B // BB
NGRP = BB // 16
VMAIN = (V // 128) * 128
VTAIL = V - VMAIN


def _sc_pooled_lookup_t(tbl_hbm, tail_hbm, idx_hbm, out_hbm,
                        slice_v, idxv0, idxv1, outv,
                        sem_s, sem_i0, sem_i1, sem_o):
    wid = lax.axis_index("s") * NC + lax.axis_index("c")
    idxv = (idxv0, idxv1)
    sem_i = (sem_i0, sem_i1)

    def idx_copy(f, blk, ib):
        return pltpu.make_async_copy(
            idx_hbm.at[f, :, pl.ds(blk * BB, BB)], idxv[ib], sem_i[ib])

    def out_copy(u):
        return pltpu.make_async_copy(outv, out_hbm.at[u], sem_o)

    _SPLITS = (0, 196 * 128, 391 * 128, 586 * 128, VMAIN)

    def slice_copies(f, d, u):
        cps = [
            pltpu.make_async_copy(
                tbl_hbm.at[f, d, pl.ds(lo, hi - lo)],
                slice_v.at[pl.ds(lo, hi - lo)], sem_s)
            for lo, hi in zip(_SPLITS[:-1], _SPLITS[1:])
        ]
        cps.append(pltpu.make_async_copy(tail_hbm.at[u],
                                         slice_v.at[pl.ds(VMAIN, 128)],
                                         sem_s))
        return cps

    def process_blocks(f):
        def blk_pair(bb_i, carry):
            for ib in range(2):
                blk = 2 * bb_i + ib
                if ib == 0:
                    idx_copy(f, blk + 1, 1).start()
                else:
                    @pl.when(bb_i < NBLK // 2 - 1)
                    def _start_next():
                        idx_copy(f, blk + 1, 0).start()
                idx_copy(f, blk, ib).wait()
                ivb = idxv[ib]
                ovb = outv

                def grp_body(grp, _g):
                    base = grp * 16
                    acc0 = plsc.load_gather(slice_v, [ivb[0, pl.ds(base, 16)]])
                    acc1 = plsc.load_gather(slice_v, [ivb[1, pl.ds(base, 16)]])
                    for l in range(2, L, 2):
                        acc0 = acc0 + plsc.load_gather(
                            slice_v, [ivb[l, pl.ds(base, 16)]])
                        acc1 = acc1 + plsc.load_gather(
                            slice_v, [ivb[l + 1, pl.ds(base, 16)]])
                    ovb[pl.ds(blk * BB + base, 16)] = acc0 + acc1
                    return _g

                lax.fori_loop(0, NGRP, grp_body, 0, unroll=4)
            return carry

        lax.fori_loop(0, NBLK // 2, blk_pair, 0, unroll=False)

    def unit_body(k, carry):
        u = wid * UNITS_W + k
        f = u // D
        d = u - f * D

        cps = slice_copies(f, d, u)
        for cp in cps:
            cp.start()
        idx_copy(f, 0, 0).start()

        @pl.when(k >= 1)
        def _drain_prev():
            out_copy(u - 1).wait()

        for cp in cps:
            cp.wait()
        process_blocks(f)
        out_copy(u).start()
        return carry

    lax.fori_loop(0, UNITS_W, unit_body, 0, unroll=False)
    out_copy(wid * UNITS_W + UNITS_W - 1).wait()


@jax.jit
def kernel(indices, tables):
    tbl_t = tables.transpose(0, 2, 1)
    idx_t = indices.astype(jnp.int32).transpose(1, 2, 0)
    tail = jnp.pad(tbl_t[:, :, VMAIN:], ((0, 0), (0, 0), (0, 128 - VTAIL)))
    tail = tail.reshape(UNITS, 128)

    mesh = plsc.VectorSubcoreMesh(core_axis_name="c", subcore_axis_name="s")
    run = functools.partial(
        pl.kernel,
        out_type=jax.ShapeDtypeStruct((UNITS, B), jnp.float32),
        mesh=mesh,
        compiler_params=pltpu.CompilerParams(needs_layout_passes=False),
        scratch_types=[
            pltpu.VMEM((VMAIN + 128,), jnp.float32),
            pltpu.VMEM((L, BB), jnp.int32),
            pltpu.VMEM((L, BB), jnp.int32),
            pltpu.VMEM((B,), jnp.float32),
            pltpu.SemaphoreType.DMA,
            pltpu.SemaphoreType.DMA,
            pltpu.SemaphoreType.DMA,
            pltpu.SemaphoreType.DMA,
        ],
    )(_sc_pooled_lookup_t)
    out_t = run(tbl_t, tail, idx_t)
    return out_t.reshape(F, D, B).transpose(2, 0, 1)
